# baseline (speedup 1.0000x reference)
import functools

import jax
import jax.numpy as jnp
from jax import lax
from jax.experimental import pallas as pl
from jax.experimental.pallas import tpu as pltpu
from jax.experimental.pallas import tpu_sc as plsc

N, D, M, S = 10000, 128, 50000, 16

NC, NS = 2, 16
CN = 8
CROWS = CN * S
NBUF = 4
TOTAL_CHUNKS = N // CN
MAX_CHUNKS = (TOTAL_CHUNKS + NS - 1) // NS
L = 16
PBLK = 8192
CRW = ((N + PBLK - 1) // PBLK) * PBLK


def _proj_body(a_ref, h_ref, p_ref):
    r = jax.lax.dot_general(a_ref[...], h_ref[...], (((1,), (1,)), ((), ())),
                            preferred_element_type=jnp.float32)
    p_ref[...] = r.reshape(PBLK // 128, 128)


def _proj(h, a):
    rows = h.shape[0]
    grid = (rows + PBLK - 1) // PBLK
    return pl.pallas_call(
        _proj_body,
        grid=(grid,),
        in_specs=[
            pl.BlockSpec((1, D), lambda i: (0, 0)),
            pl.BlockSpec((PBLK, D), lambda i: (i, 0)),
        ],
        out_specs=pl.BlockSpec((PBLK // 128, 128), lambda i: (i, 0)),
        out_shape=jax.ShapeDtypeStruct((grid * (PBLK // 128), 128),
                                       jnp.float32),
    )(a, h)


def _sc_attention(h1, h2, n1f, n2f, p1, p2, c1, c2):
    mesh = plsc.VectorSubcoreMesh(core_axis_name="c", subcore_axis_name="s")

    @functools.partial(
        pl.kernel,
        mesh=mesh,
        out_type=(
            jax.ShapeDtypeStruct((N, D), jnp.float32),
            jax.ShapeDtypeStruct((N, D), jnp.float32),
        ),
        scratch_types=[
            [pltpu.VMEM((CROWS,), jnp.int32) for _ in range(NBUF)],
            [pltpu.VMEM((CROWS, D), jnp.float32) for _ in range(NBUF)],
            [pltpu.VMEM((CROWS,), jnp.float32) for _ in range(NBUF)],
            [pltpu.VMEM((CN, D), jnp.float32) for _ in range(NBUF)],
            pltpu.VMEM((CRW,), jnp.float32),
            [pltpu.SemaphoreType.DMA for _ in range(NBUF)],
            [pltpu.SemaphoreType.DMA for _ in range(NBUF)],
            [pltpu.SemaphoreType.DMA for _ in range(NBUF)],
        ],
    )
    def body(h1_hbm, h2_hbm, n1_hbm, n2_hbm, p1_hbm, p2_hbm, c1_hbm, c2_hbm,
             e1_hbm, e2_hbm, idx_v, rows_v, pch_v, e_v, c_v, gsem, psem, wsem):
        c = lax.axis_index("c")
        s = lax.axis_index("s")
        nchunk = (TOTAL_CHUNKS - s + NS - 1) // NS
        iota = lax.iota(jnp.int32, L)
        perms = [(iota + sh) & (L - 1) for sh in (8, 4, 2, 1)]
        lanes = [iota * 0 + sn for sn in range(S)]
        dnums = lax.GatherDimensionNumbers(
            offset_dims=(), collapsed_slice_dims=(0,), start_index_map=(0,))

        def take(v, p):
            return lax.gather(v, p[:, None], dnums, (1,),
                              mode=lax.GatherScatterMode.PROMISE_IN_BOUNDS)

        def vsum(v):
            for p in perms:
                v = v + take(v, p)
            return v

        def run(table, neif, p_hbm, c_hbm, e_out):
            pltpu.sync_copy(c_hbm, c_v)

            def node_base(k):
                return pl.multiple_of((s + k * NS) * CN, 8)

            def fire(k, b):
                @pl.when(k < nchunk)
                def _():
                    nb = node_base(k)
                    pltpu.sync_copy(neif.at[pl.ds(nb * S, CROWS)], idx_v[b])
                    pltpu.async_copy(table.at[idx_v[b]], rows_v[b], gsem[b])
                    pltpu.async_copy(p_hbm.at[idx_v[b]], pch_v[b], psem[b])

            def compute(k, b):
                rows = rows_v[b]
                pch = pch_v[b]
                eb = e_v[b]
                nb = node_base(k)

                def node(i, carry):
                    cs = c_v[pl.ds(nb + i, L)][0]
                    d = pch[pl.ds(i * S, L)] + cs
                    d = jnp.where(d > 0.0, d, 0.01 * d)
                    ex = jnp.exp(d)
                    sinv = 1.0 / vsum(ex)

                    eacc = [jnp.zeros((L,), jnp.float32) for _ in range(8)]
                    for sn in range(S):
                        r = i * S + sn
                        exs = take(ex, lanes[sn])
                        for j in range(8):
                            eacc[j] = eacc[j] + exs * rows[r, pl.ds(L * j, L)]
                    for j in range(8):
                        e = eacc[j] * sinv
                        e = jnp.where(e > 0.0, e,
                                      jnp.exp(jnp.minimum(e, 0.0)) - 1.0)
                        eb[i, pl.ds(L * j, L)] = e
                    return carry

                lax.fori_loop(0, CN, node, 0)

            def stage(k, b):
                @pl.when(k < nchunk)
                def _():
                    pltpu.make_async_copy(
                        table.at[idx_v[b]], rows_v[b], gsem[b]).wait()
                    pltpu.make_async_copy(
                        p_hbm.at[idx_v[b]], pch_v[b], psem[b]).wait()

                    @pl.when(k >= NBUF)
                    def _():
                        pltpu.make_async_copy(
                            e_v[b], e_out.at[pl.ds(0, CN), :], wsem[b]).wait()

                    compute(k, b)
                    nb = node_base(k)
                    pltpu.async_copy(e_v[b], e_out.at[pl.ds(nb, CN), :],
                                     wsem[b])
                    fire(k + NBUF, b)

            for b in range(NBUF):
                fire(b, b)

            def ring(kk, carry):
                for b in range(NBUF):
                    stage(kk * NBUF + b, b)
                return carry

            lax.fori_loop(0, (MAX_CHUNKS + NBUF - 1) // NBUF, ring, 0)

            for b in range(NBUF):
                pltpu.make_async_copy(
                    e_v[b], e_out.at[pl.ds(0, CN), :], wsem[b]).wait()

        @pl.when(c == 0)
        def _():
            run(h1_hbm, n1_hbm, p1_hbm, c1_hbm, e1_hbm)

        @pl.when(c == 1)
        def _():
            run(h2_hbm, n2_hbm, p2_hbm, c2_hbm, e2_hbm)

    return body(h1, h2, n1f, n2f, p1, p2, c1, c2)


BN = 2000
GRID = N // BN


def _fc_sp_body(e1_ref, e2_ref, fcw_ref, fcb_ref, sp_ref):
    @pl.when(pl.program_id(0) == 0)
    def _():
        sp_ref[...] = jnp.zeros_like(sp_ref)

    fcw = fcw_ref[...]
    fcb = fcb_ref[...]
    for v, e_ref in enumerate((e1_ref, e2_ref)):
        t = jnp.tanh(
            jax.lax.dot_general(e_ref[...], fcw, (((1,), (1,)), ((), ())),
                                preferred_element_type=jnp.float32) + fcb)
        sp_ref[v:v + 1, :] += jnp.sum(t, axis=0, keepdims=True)


def _fc_sp(e1, e2, fc_w, fc_b):
    full = lambda shape: pl.BlockSpec(shape, lambda i: tuple(0 for _ in shape))
    return pl.pallas_call(
        _fc_sp_body,
        grid=(GRID,),
        in_specs=[
            pl.BlockSpec((BN, D), lambda i: (i, 0)),
            pl.BlockSpec((BN, D), lambda i: (i, 0)),
            full((D, D)),
            full((1, D)),
        ],
        out_specs=full((2, D)),
        out_shape=jax.ShapeDtypeStruct((2, D), jnp.float32),
    )(e1, e2, fc_w, fc_b)


def _combine_body(e1_ref, e2_ref, sp_ref, ai_ref, z_ref):
    b = jnp.sum(ai_ref[...] * sp_ref[...], axis=1, keepdims=True) / N
    m = jnp.max(b, axis=0, keepdims=True)
    ex = jnp.exp(b - m)
    beta = ex / jnp.sum(ex, axis=0, keepdims=True)
    z_ref[...] = (e1_ref[...] * beta[0:1, 0:1]
                  + e2_ref[...] * beta[1:2, 0:1])


def _combine(e1, e2, sp, att_inter):
    return pl.pallas_call(
        _combine_body,
        out_shape=jax.ShapeDtypeStruct((N, D), jnp.float32),
    )(e1, e2, sp, att_inter)


def kernel(h0, h1, h2, nei1, nei2, att_intra1, att_intra2, fc_w, fc_b, att_inter):
    a1r, a1n = att_intra1[:, :D], att_intra1[:, D:]
    a2r, a2n = att_intra2[:, :D], att_intra2[:, D:]
    p1 = _proj(h1, a1n)
    p2 = _proj(h2, a2n)
    c1 = _proj(h0, a1r)
    c2 = _proj(h0, a2r)
    e1, e2 = _sc_attention(h1, h2, nei1.reshape(-1), nei2.reshape(-1),
                           p1.reshape(-1), p2.reshape(-1),
                           c1.reshape(-1), c2.reshape(-1))
    sp = _fc_sp(e1, e2, fc_w, fc_b.reshape(1, D))
    return _combine(e1, e2, sp, att_inter)


# speedup vs baseline: 1.1245x; 1.1245x over previous
import functools

import jax
import jax.numpy as jnp
from jax import lax
from jax.experimental import pallas as pl
from jax.experimental.pallas import tpu as pltpu
from jax.experimental.pallas import tpu_sc as plsc

N, D, M, S = 10000, 128, 50000, 16

NC, NS = 2, 16
NW = NC * NS
CN = 16
CROWS = CN * S
NBUF = 3
TOTAL_CHUNKS = N // CN
MAX_CHUNKS = (TOTAL_CHUNKS + NW - 1) // NW
L = 16
PBLK = 8192
CRW = ((N + PBLK - 1) // PBLK) * PBLK


def _proj_body(a_ref, h_ref, p_ref):
    r = jax.lax.dot_general(a_ref[...], h_ref[...], (((1,), (1,)), ((), ())),
                            preferred_element_type=jnp.float32)
    p_ref[...] = r.reshape(PBLK // 128, 128)


def _proj(h, a):
    rows = h.shape[0]
    grid = (rows + PBLK - 1) // PBLK
    return pl.pallas_call(
        _proj_body,
        grid=(grid,),
        in_specs=[
            pl.BlockSpec((1, D), lambda i: (0, 0)),
            pl.BlockSpec((PBLK, D), lambda i: (i, 0)),
        ],
        out_specs=pl.BlockSpec((PBLK // 128, 128), lambda i: (i, 0)),
        out_shape=jax.ShapeDtypeStruct((grid * (PBLK // 128), 128),
                                       jnp.float32),
    )(a, h)


def _sc_attention(table_in, nf, p, c_in):
    mesh = plsc.VectorSubcoreMesh(core_axis_name="c", subcore_axis_name="s")

    @functools.partial(
        pl.kernel,
        mesh=mesh,
        out_type=jax.ShapeDtypeStruct((N, D), jnp.float32),
        scratch_types=[
            [pltpu.VMEM((CROWS,), jnp.int32) for _ in range(NBUF)],
            [pltpu.VMEM((CROWS, D), jnp.float32) for _ in range(NBUF)],
            [pltpu.VMEM((CROWS,), jnp.float32) for _ in range(NBUF)],
            [pltpu.VMEM((CN, D), jnp.float32) for _ in range(NBUF)],
            pltpu.VMEM((CRW,), jnp.float32),
            [pltpu.SemaphoreType.DMA for _ in range(NBUF)],
            [pltpu.SemaphoreType.DMA for _ in range(NBUF)],
            [pltpu.SemaphoreType.DMA for _ in range(NBUF)],
        ],
    )
    def body(table, neif, p_hbm, c_hbm, e_out,
             idx_v, rows_v, pch_v, e_v, c_v, gsem, psem, wsem):
        w = lax.axis_index("s") * NC + lax.axis_index("c")
        nchunk = (TOTAL_CHUNKS - w + NW - 1) // NW
        iota = lax.iota(jnp.int32, L)
        perms = [(iota + sh) & (L - 1) for sh in (8, 4, 2, 1)]
        lanes = [iota * 0 + sn for sn in range(S)]
        dnums = lax.GatherDimensionNumbers(
            offset_dims=(), collapsed_slice_dims=(0,), start_index_map=(0,))

        def take(v, pm):
            return lax.gather(v, pm[:, None], dnums, (1,),
                              mode=lax.GatherScatterMode.PROMISE_IN_BOUNDS)

        def vsum(v):
            for pm in perms:
                v = v + take(v, pm)
            return v

        pltpu.sync_copy(c_hbm, c_v)

        def node_base(k):
            return pl.multiple_of((w + k * NW) * CN, 8)

        def fire(k, b):
            @pl.when(k < nchunk)
            def _():
                nb = node_base(k)
                pltpu.sync_copy(neif.at[pl.ds(nb * S, CROWS)], idx_v[b])
                pltpu.async_copy(table.at[idx_v[b]], rows_v[b], gsem[b])
                pltpu.async_copy(p_hbm.at[idx_v[b]], pch_v[b], psem[b])

        def compute(k, b):
            rows = rows_v[b]
            pch = pch_v[b]
            eb = e_v[b]
            nb = node_base(k)

            def node(i, carry):
                cs = c_v[pl.ds(nb + i, L)][0]
                d = pch[pl.ds(i * S, L)] + cs
                d = jnp.where(d > 0.0, d, 0.01 * d)
                ex = jnp.exp(d)
                sinv = 1.0 / vsum(ex)

                eacc = [jnp.zeros((L,), jnp.float32) for _ in range(8)]
                for sn in range(S):
                    r = i * S + sn
                    exs = take(ex, lanes[sn])
                    for j in range(8):
                        eacc[j] = eacc[j] + exs * rows[r, pl.ds(L * j, L)]
                for j in range(8):
                    e = eacc[j] * sinv
                    e = jnp.where(e > 0.0, e,
                                  jnp.exp(jnp.minimum(e, 0.0)) - 1.0)
                    eb[i, pl.ds(L * j, L)] = e
                return carry

            lax.fori_loop(0, CN, node, 0)

        def stage(k, b):
            @pl.when(k < nchunk)
            def _():
                pltpu.make_async_copy(
                    table.at[idx_v[b]], rows_v[b], gsem[b]).wait()
                pltpu.make_async_copy(
                    p_hbm.at[idx_v[b]], pch_v[b], psem[b]).wait()

                @pl.when(k >= NBUF)
                def _():
                    pltpu.make_async_copy(
                        e_v[b], e_out.at[pl.ds(0, CN), :], wsem[b]).wait()

                compute(k, b)
                nb = node_base(k)
                pltpu.async_copy(e_v[b], e_out.at[pl.ds(nb, CN), :], wsem[b])
                fire(k + NBUF, b)

        for b in range(NBUF):
            fire(b, b)

        def ring(kk, carry):
            for b in range(NBUF):
                stage(kk * NBUF + b, b)
            return carry

        lax.fori_loop(0, (MAX_CHUNKS + NBUF - 1) // NBUF, ring, 0)

        for b in range(NBUF):
            pltpu.make_async_copy(
                e_v[b], e_out.at[pl.ds(0, CN), :], wsem[b]).wait()

    return body(table_in, nf, p, c_in)


BN = 2000
GRID = N // BN


def _fc_sp_body(e_ref, fcw_ref, fcb_ref, sp_ref):
    @pl.when(pl.program_id(0) == 0)
    def _():
        sp_ref[...] = jnp.zeros_like(sp_ref)

    t = jnp.tanh(
        jax.lax.dot_general(e_ref[...], fcw_ref[...], (((1,), (1,)), ((), ())),
                            preferred_element_type=jnp.float32) + fcb_ref[...])
    sp_ref[...] += jnp.sum(t, axis=0, keepdims=True)


def _fc_sp(e, fc_w, fc_b):
    full = lambda shape: pl.BlockSpec(shape, lambda i: tuple(0 for _ in shape))
    return pl.pallas_call(
        _fc_sp_body,
        grid=(GRID,),
        in_specs=[
            pl.BlockSpec((BN, D), lambda i: (i, 0)),
            full((D, D)),
            full((1, D)),
        ],
        out_specs=full((1, D)),
        out_shape=jax.ShapeDtypeStruct((1, D), jnp.float32),
    )(e, fc_w, fc_b)


def _combine_body(e1_ref, e2_ref, sp1_ref, sp2_ref, ai_ref, z_ref):
    ai = ai_ref[...]
    b1 = jnp.sum(ai * sp1_ref[...], axis=1, keepdims=True) / N
    b2 = jnp.sum(ai * sp2_ref[...], axis=1, keepdims=True) / N
    m = jnp.maximum(b1, b2)
    x1 = jnp.exp(b1 - m)
    x2 = jnp.exp(b2 - m)
    tot = x1 + x2
    z_ref[...] = (e1_ref[...] * (x1 / tot) + e2_ref[...] * (x2 / tot))


def _combine(e1, e2, sp1, sp2, att_inter):
    return pl.pallas_call(
        _combine_body,
        out_shape=jax.ShapeDtypeStruct((N, D), jnp.float32),
    )(e1, e2, sp1, sp2, att_inter)


def kernel(h0, h1, h2, nei1, nei2, att_intra1, att_intra2, fc_w, fc_b, att_inter):
    a1r, a1n = att_intra1[:, :D], att_intra1[:, D:]
    a2r, a2n = att_intra2[:, :D], att_intra2[:, D:]
    fcb = fc_b.reshape(1, D)
    p1 = _proj(h1, a1n)
    c1 = _proj(h0, a1r)
    e1 = _sc_attention(h1, nei1.reshape(-1), p1.reshape(-1), c1.reshape(-1))
    p2 = _proj(h2, a2n)
    c2 = _proj(h0, a2r)
    e2 = _sc_attention(h2, nei2.reshape(-1), p2.reshape(-1), c2.reshape(-1))
    sp1 = _fc_sp(e1, fc_w, fcb)
    sp2 = _fc_sp(e2, fc_w, fcb)
    return _combine(e1, e2, sp1, sp2, att_inter)


# speedup vs baseline: 1.1291x; 1.0041x over previous
import functools

import jax
import jax.numpy as jnp
from jax import lax
from jax.experimental import pallas as pl
from jax.experimental.pallas import tpu as pltpu
from jax.experimental.pallas import tpu_sc as plsc

N, D, M, S = 10000, 128, 50000, 16

NC, NS = 2, 16
NW = NC * NS
CN = 16
CROWS = CN * S
NBUF = 3
TOTAL_CHUNKS = N // CN
MAX_CHUNKS = (TOTAL_CHUNKS + NW - 1) // NW
L = 16
PBLK = 8192
CRW = ((N + PBLK - 1) // PBLK) * PBLK


def _proj_body(a_ref, h_ref, p_ref):
    r = jax.lax.dot_general(a_ref[...], h_ref[...], (((1,), (1,)), ((), ())),
                            preferred_element_type=jnp.float32)
    p_ref[...] = r.reshape(PBLK)


def _proj(h, a):
    rows = h.shape[0]
    grid = (rows + PBLK - 1) // PBLK
    return pl.pallas_call(
        _proj_body,
        grid=(grid,),
        in_specs=[
            pl.BlockSpec((1, D), lambda i: (0, 0)),
            pl.BlockSpec((PBLK, D), lambda i: (i, 0)),
        ],
        out_specs=pl.BlockSpec((PBLK,), lambda i: (i,)),
        out_shape=jax.ShapeDtypeStruct((grid * PBLK,), jnp.float32),
    )(a, h)


def _projc_body(a1_ref, a2_ref, h_ref, c1_ref, c2_ref):
    h = h_ref[...]
    for a_ref, c_ref in ((a1_ref, c1_ref), (a2_ref, c2_ref)):
        r = jax.lax.dot_general(a_ref[...], h, (((1,), (1,)), ((), ())),
                                preferred_element_type=jnp.float32)
        c_ref[...] = r.reshape(PBLK)


def _projc(h, a1, a2):
    rows = h.shape[0]
    grid = (rows + PBLK - 1) // PBLK
    out = jax.ShapeDtypeStruct((grid * PBLK,), jnp.float32)
    return pl.pallas_call(
        _projc_body,
        grid=(grid,),
        in_specs=[
            pl.BlockSpec((1, D), lambda i: (0, 0)),
            pl.BlockSpec((1, D), lambda i: (0, 0)),
            pl.BlockSpec((PBLK, D), lambda i: (i, 0)),
        ],
        out_specs=[pl.BlockSpec((PBLK,), lambda i: (i,)),
                   pl.BlockSpec((PBLK,), lambda i: (i,))],
        out_shape=[out, out],
    )(a1, a2, h)


def _sc_attention(table_in, nf, p, c_in):
    mesh = plsc.VectorSubcoreMesh(core_axis_name="c", subcore_axis_name="s")

    @functools.partial(
        pl.kernel,
        mesh=mesh,
        out_type=jax.ShapeDtypeStruct((N, D), jnp.float32),
        scratch_types=[
            [pltpu.VMEM((CROWS,), jnp.int32) for _ in range(NBUF)],
            [pltpu.VMEM((CROWS, D), jnp.float32) for _ in range(NBUF)],
            [pltpu.VMEM((CROWS,), jnp.float32) for _ in range(NBUF)],
            [pltpu.VMEM((CN, D), jnp.float32) for _ in range(NBUF)],
            pltpu.VMEM((CRW,), jnp.float32),
            [pltpu.SemaphoreType.DMA for _ in range(NBUF)],
            [pltpu.SemaphoreType.DMA for _ in range(NBUF)],
            [pltpu.SemaphoreType.DMA for _ in range(NBUF)],
        ],
    )
    def body(table, neif, p_hbm, c_hbm, e_out,
             idx_v, rows_v, pch_v, e_v, c_v, gsem, psem, wsem):
        w = lax.axis_index("s") * NC + lax.axis_index("c")
        nchunk = (TOTAL_CHUNKS - w + NW - 1) // NW
        iota = lax.iota(jnp.int32, L)
        perms = [(iota + sh) & (L - 1) for sh in (8, 4, 2, 1)]
        lanes = [iota * 0 + sn for sn in range(S)]
        dnums = lax.GatherDimensionNumbers(
            offset_dims=(), collapsed_slice_dims=(0,), start_index_map=(0,))

        def take(v, pm):
            return lax.gather(v, pm[:, None], dnums, (1,),
                              mode=lax.GatherScatterMode.PROMISE_IN_BOUNDS)

        def vsum(v):
            for pm in perms:
                v = v + take(v, pm)
            return v

        pltpu.sync_copy(c_hbm, c_v)

        def node_base(k):
            return pl.multiple_of((w + k * NW) * CN, 8)

        def fire(k, b):
            @pl.when(k < nchunk)
            def _():
                nb = node_base(k)
                pltpu.sync_copy(neif.at[pl.ds(nb * S, CROWS)], idx_v[b])
                pltpu.async_copy(table.at[idx_v[b]], rows_v[b], gsem[b])
                pltpu.async_copy(p_hbm.at[idx_v[b]], pch_v[b], psem[b])

        def compute(k, b):
            rows = rows_v[b]
            pch = pch_v[b]
            eb = e_v[b]
            nb = node_base(k)

            def node(i, carry):
                cs = c_v[pl.ds(nb + i, L)][0]
                d = pch[pl.ds(i * S, L)] + cs
                d = jnp.where(d > 0.0, d, 0.01 * d)
                ex = jnp.exp(d)
                sinv = 1.0 / vsum(ex)

                eacc = [jnp.zeros((L,), jnp.float32) for _ in range(8)]
                for sn in range(S):
                    r = i * S + sn
                    exs = take(ex, lanes[sn])
                    for j in range(8):
                        eacc[j] = eacc[j] + exs * rows[r, pl.ds(L * j, L)]
                for j in range(8):
                    e = eacc[j] * sinv
                    e = jnp.where(e > 0.0, e,
                                  jnp.exp(jnp.minimum(e, 0.0)) - 1.0)
                    eb[i, pl.ds(L * j, L)] = e
                return carry

            lax.fori_loop(0, CN, node, 0)

        def stage(k, b):
            @pl.when(k < nchunk)
            def _():
                pltpu.make_async_copy(
                    table.at[idx_v[b]], rows_v[b], gsem[b]).wait()
                pltpu.make_async_copy(
                    p_hbm.at[idx_v[b]], pch_v[b], psem[b]).wait()

                @pl.when(k >= NBUF)
                def _():
                    pltpu.make_async_copy(
                        e_v[b], e_out.at[pl.ds(0, CN), :], wsem[b]).wait()

                compute(k, b)
                nb = node_base(k)
                pltpu.async_copy(e_v[b], e_out.at[pl.ds(nb, CN), :], wsem[b])
                fire(k + NBUF, b)

        for b in range(NBUF):
            fire(b, b)

        def ring(kk, carry):
            for b in range(NBUF):
                stage(kk * NBUF + b, b)
            return carry

        lax.fori_loop(0, (MAX_CHUNKS + NBUF - 1) // NBUF, ring, 0)

        for b in range(NBUF):
            pltpu.make_async_copy(
                e_v[b], e_out.at[pl.ds(0, CN), :], wsem[b]).wait()

    return body(table_in, nf, p, c_in)


BN = 2000
GRID = N // BN


def _fc_sp_body(e_ref, fcw_ref, fcb_ref, sp_ref):
    @pl.when(pl.program_id(0) == 0)
    def _():
        sp_ref[...] = jnp.zeros_like(sp_ref)

    t = jnp.tanh(
        jax.lax.dot_general(e_ref[...], fcw_ref[...], (((1,), (1,)), ((), ())),
                            preferred_element_type=jnp.float32) + fcb_ref[...])
    sp_ref[...] += jnp.sum(t, axis=0, keepdims=True)


def _fc_sp(e, fc_w, fc_b):
    full = lambda shape: pl.BlockSpec(shape, lambda i: tuple(0 for _ in shape))
    return pl.pallas_call(
        _fc_sp_body,
        grid=(GRID,),
        in_specs=[
            pl.BlockSpec((BN, D), lambda i: (i, 0)),
            full((D, D)),
            full((1, D)),
        ],
        out_specs=full((1, D)),
        out_shape=jax.ShapeDtypeStruct((1, D), jnp.float32),
    )(e, fc_w, fc_b)


def _combine_body(e1_ref, e2_ref, sp1_ref, sp2_ref, ai_ref, z_ref):
    ai = ai_ref[...]
    b1 = jnp.sum(ai * sp1_ref[...], axis=1, keepdims=True) / N
    b2 = jnp.sum(ai * sp2_ref[...], axis=1, keepdims=True) / N
    m = jnp.maximum(b1, b2)
    x1 = jnp.exp(b1 - m)
    x2 = jnp.exp(b2 - m)
    tot = x1 + x2
    z_ref[...] = (e1_ref[...] * (x1 / tot) + e2_ref[...] * (x2 / tot))


def _combine(e1, e2, sp1, sp2, att_inter):
    return pl.pallas_call(
        _combine_body,
        out_shape=jax.ShapeDtypeStruct((N, D), jnp.float32),
    )(e1, e2, sp1, sp2, att_inter)


def kernel(h0, h1, h2, nei1, nei2, att_intra1, att_intra2, fc_w, fc_b, att_inter):
    a1r, a1n = att_intra1[:, :D], att_intra1[:, D:]
    a2r, a2n = att_intra2[:, :D], att_intra2[:, D:]
    fcb = fc_b.reshape(1, D)
    c1, c2 = _projc(h0, a1r, a2r)
    p1 = _proj(h1, a1n)
    e1 = _sc_attention(h1, nei1.reshape(-1), p1, c1)
    p2 = _proj(h2, a2n)
    e2 = _sc_attention(h2, nei2.reshape(-1), p2, c2)
    sp1 = _fc_sp(e1, fc_w, fcb)
    sp2 = _fc_sp(e2, fc_w, fcb)
    return _combine(e1, e2, sp1, sp2, att_inter)


# speedup vs baseline: 1.1687x; 1.0351x over previous
import functools

import jax
import jax.numpy as jnp
from jax import lax
from jax.experimental import pallas as pl
from jax.experimental.pallas import tpu as pltpu
from jax.experimental.pallas import tpu_sc as plsc

N, D, M, S = 10000, 128, 50000, 16

NC, NS = 2, 16
NW = NC * NS
CN = 16
CROWS = CN * S
NBUF = 3
TOTAL_CHUNKS = N // CN
MAX_CHUNKS = (TOTAL_CHUNKS + NW - 1) // NW
L = 16
PBLK = 8192
CRW = ((N + PBLK - 1) // PBLK) * PBLK


def _proj_body(a_ref, h_ref, p_ref):
    r = jax.lax.dot_general(a_ref[...], h_ref[...], (((1,), (1,)), ((), ())),
                            preferred_element_type=jnp.float32)
    p_ref[...] = r.reshape(PBLK)


def _proj(h, a):
    rows = h.shape[0]
    grid = (rows + PBLK - 1) // PBLK
    return pl.pallas_call(
        _proj_body,
        grid=(grid,),
        in_specs=[
            pl.BlockSpec((1, D), lambda i: (0, 0)),
            pl.BlockSpec((PBLK, D), lambda i: (i, 0)),
        ],
        out_specs=pl.BlockSpec((PBLK,), lambda i: (i,)),
        out_shape=jax.ShapeDtypeStruct((grid * PBLK,), jnp.float32),
    )(a, h)


def _projc_body(a1_ref, a2_ref, h_ref, c1_ref, c2_ref):
    h = h_ref[...]
    for a_ref, c_ref in ((a1_ref, c1_ref), (a2_ref, c2_ref)):
        r = jax.lax.dot_general(a_ref[...], h, (((1,), (1,)), ((), ())),
                                preferred_element_type=jnp.float32)
        c_ref[...] = r.reshape(PBLK)


def _projc(h, a1, a2):
    rows = h.shape[0]
    grid = (rows + PBLK - 1) // PBLK
    out = jax.ShapeDtypeStruct((grid * PBLK,), jnp.float32)
    return pl.pallas_call(
        _projc_body,
        grid=(grid,),
        in_specs=[
            pl.BlockSpec((1, D), lambda i: (0, 0)),
            pl.BlockSpec((1, D), lambda i: (0, 0)),
            pl.BlockSpec((PBLK, D), lambda i: (i, 0)),
        ],
        out_specs=[pl.BlockSpec((PBLK,), lambda i: (i,)),
                   pl.BlockSpec((PBLK,), lambda i: (i,))],
        out_shape=[out, out],
    )(a1, a2, h)


def _sc_attention(table_in, nf, p, c_in):
    mesh = plsc.VectorSubcoreMesh(core_axis_name="c", subcore_axis_name="s")

    @functools.partial(
        pl.kernel,
        mesh=mesh,
        out_type=jax.ShapeDtypeStruct((N, D), jnp.float32),
        scratch_types=[
            [pltpu.VMEM((CROWS,), jnp.int32) for _ in range(NBUF)],
            [pltpu.VMEM((CROWS, D), jnp.float32) for _ in range(NBUF)],
            [pltpu.VMEM((CROWS,), jnp.float32) for _ in range(NBUF)],
            [pltpu.VMEM((CN, D), jnp.float32) for _ in range(NBUF)],
            pltpu.VMEM((CRW,), jnp.float32),
            [pltpu.SemaphoreType.DMA for _ in range(NBUF)],
            [pltpu.SemaphoreType.DMA for _ in range(NBUF)],
            [pltpu.SemaphoreType.DMA for _ in range(NBUF)],
        ],
    )
    def body(table, neif, p_hbm, c_hbm, e_out,
             idx_v, rows_v, pch_v, e_v, c_v, gsem, psem, wsem):
        w = lax.axis_index("s") * NC + lax.axis_index("c")
        nchunk = (TOTAL_CHUNKS - w + NW - 1) // NW
        iota = lax.iota(jnp.int32, L)
        perms = [(iota + sh) & (L - 1) for sh in (8, 4, 2, 1)]
        lanes = [iota * 0 + sn for sn in range(S)]
        dnums = lax.GatherDimensionNumbers(
            offset_dims=(), collapsed_slice_dims=(0,), start_index_map=(0,))

        def take(v, pm):
            return lax.gather(v, pm[:, None], dnums, (1,),
                              mode=lax.GatherScatterMode.PROMISE_IN_BOUNDS)

        def vsum(v):
            for pm in perms:
                v = v + take(v, pm)
            return v

        pltpu.sync_copy(c_hbm, c_v)

        def node_base(k):
            return pl.multiple_of((w + k * NW) * CN, 8)

        def fire(k, b):
            @pl.when(k < nchunk)
            def _():
                nb = node_base(k)
                pltpu.sync_copy(neif.at[pl.ds(nb * S, CROWS)], idx_v[b])
                pltpu.async_copy(table.at[idx_v[b]], rows_v[b], gsem[b])
                pltpu.async_copy(p_hbm.at[idx_v[b]], pch_v[b], psem[b])

        def compute(k, b):
            rows = rows_v[b]
            pch = pch_v[b]
            eb = e_v[b]
            nb = node_base(k)

            def node(i, carry):
                cs = c_v[pl.ds(nb + i, L)][0]
                d = pch[pl.ds(i * S, L)] + cs
                d = jnp.where(d > 0.0, d, 0.01 * d)
                ex = jnp.exp(d)
                sinv = 1.0 / vsum(ex)

                eacc = [jnp.zeros((L,), jnp.float32) for _ in range(8)]
                for sn in range(S):
                    r = i * S + sn
                    exs = take(ex, lanes[sn])
                    for j in range(8):
                        eacc[j] = eacc[j] + exs * rows[r, pl.ds(L * j, L)]
                for j in range(8):
                    e = eacc[j] * sinv
                    e = jnp.where(e > 0.0, e,
                                  jnp.exp(jnp.minimum(e, 0.0)) - 1.0)
                    eb[i, pl.ds(L * j, L)] = e
                return carry

            lax.fori_loop(0, CN, node, 0)

        def stage(k, b):
            @pl.when(k < nchunk)
            def _():
                pltpu.make_async_copy(
                    table.at[idx_v[b]], rows_v[b], gsem[b]).wait()
                pltpu.make_async_copy(
                    p_hbm.at[idx_v[b]], pch_v[b], psem[b]).wait()

                @pl.when(k >= NBUF)
                def _():
                    pltpu.make_async_copy(
                        e_v[b], e_out.at[pl.ds(0, CN), :], wsem[b]).wait()

                compute(k, b)
                nb = node_base(k)
                pltpu.async_copy(e_v[b], e_out.at[pl.ds(nb, CN), :], wsem[b])
                fire(k + NBUF, b)

        for b in range(NBUF):
            fire(b, b)

        def ring(kk, carry):
            for b in range(NBUF):
                stage(kk * NBUF + b, b)
            return carry

        lax.fori_loop(0, (MAX_CHUNKS + NBUF - 1) // NBUF, ring, 0)

        for b in range(NBUF):
            pltpu.make_async_copy(
                e_v[b], e_out.at[pl.ds(0, CN), :], wsem[b]).wait()

    return body(table_in, nf, p, c_in)


BN = 2000
GRID = N // BN


def _fc_sp_body(e_ref, fcw_ref, fcb_ref, sp_ref):
    @pl.when(pl.program_id(0) == 0)
    def _():
        sp_ref[...] = jnp.zeros_like(sp_ref)

    t = jnp.tanh(
        jax.lax.dot_general(e_ref[...], fcw_ref[...], (((1,), (1,)), ((), ())),
                            preferred_element_type=jnp.float32) + fcb_ref[...])
    sp_ref[...] += jnp.sum(t, axis=0, keepdims=True)


def _fc_sp(e, fc_w, fc_b):
    full = lambda shape: pl.BlockSpec(shape, lambda i: tuple(0 for _ in shape))
    return pl.pallas_call(
        _fc_sp_body,
        grid=(GRID,),
        in_specs=[
            pl.BlockSpec((BN, D), lambda i: (i, 0)),
            full((D, D)),
            full((1, D)),
        ],
        out_specs=full((1, D)),
        out_shape=jax.ShapeDtypeStruct((1, D), jnp.float32),
    )(e, fc_w, fc_b)


def _combine_body(e1_ref, e2_ref, sp1_ref, fcw_ref, fcb_ref, ai_ref, z_ref):
    e2 = e2_ref[...]
    t2 = jnp.tanh(
        jax.lax.dot_general(e2, fcw_ref[...], (((1,), (1,)), ((), ())),
                            preferred_element_type=jnp.float32) + fcb_ref[...])
    sp2 = jnp.sum(t2, axis=0, keepdims=True)
    ai = ai_ref[...]
    b1 = jnp.sum(ai * sp1_ref[...], axis=1, keepdims=True) / N
    b2 = jnp.sum(ai * sp2, axis=1, keepdims=True) / N
    m = jnp.maximum(b1, b2)
    x1 = jnp.exp(b1 - m)
    x2 = jnp.exp(b2 - m)
    tot = x1 + x2
    z_ref[...] = (e1_ref[...] * (x1 / tot) + e2 * (x2 / tot))


def _combine(e1, e2, sp1, fc_w, fc_b, att_inter):
    return pl.pallas_call(
        _combine_body,
        out_shape=jax.ShapeDtypeStruct((N, D), jnp.float32),
    )(e1, e2, sp1, fc_w, fc_b, att_inter)


def kernel(h0, h1, h2, nei1, nei2, att_intra1, att_intra2, fc_w, fc_b, att_inter):
    a1r, a1n = att_intra1[:, :D], att_intra1[:, D:]
    a2r, a2n = att_intra2[:, :D], att_intra2[:, D:]
    fcb = fc_b.reshape(1, D)
    c1, c2 = _projc(h0, a1r, a2r)
    p1 = _proj(h1, a1n)
    e1 = _sc_attention(h1, nei1.reshape(-1), p1, c1)
    p2 = _proj(h2, a2n)
    e2 = _sc_attention(h2, nei2.reshape(-1), p2, c2)
    sp1 = _fc_sp(e1, fc_w, fcb)
    return _combine(e1, e2, sp1, fc_w, fcb, att_inter)
